# Initial kernel scaffold; baseline (speedup 1.0000x reference)
import jax
import jax.numpy as jnp
from jax.experimental import pallas as pl


def kernel(params, src_has_lab, dst_has_lab, src_rev_has_lab, dst_rev_has_lab, src_has_diag, dst_has_diag, src_rev_has_diag, dst_rev_has_diag, src_has_med, dst_has_med, src_rev_has_med, dst_rev_has_med):
    raise NotImplementedError("write your pallas kernel here")


# speedup vs baseline: 1.2792x; 1.2792x over previous
import functools

import jax
import jax.numpy as jnp
from jax import lax
from jax.experimental import pallas as pl
from jax.experimental.pallas import tpu as pltpu
from jax.experimental.pallas import tpu_sc as plsc

D = 128
NE = 100000
NC, NS, L = 2, 16, 16
NW = NC * NS
EB = 80
NBATCH = NE // EB
MAXB = (NBATCH + NW - 1) // NW
BN_EPS = 1e-5

REL_NAMES = ["has_lab", "rev_has_lab", "has_diag", "rev_has_diag",
             "has_med", "rev_has_med"]
REL_SRC = {"has_lab": "patient", "rev_has_lab": "lab",
           "has_diag": "patient", "rev_has_diag": "diagnosis",
           "has_med": "patient", "rev_has_med": "medication"}
REL_DST = {"has_lab": "lab", "rev_has_lab": "patient",
           "has_diag": "diagnosis", "rev_has_diag": "patient",
           "has_med": "medication", "rev_has_med": "patient"}
NT_ORDER = ["patient", "lab", "diagnosis", "medication"]


def _row_block(n):
    return 2000 if n % 2000 == 0 and n > 5000 else n


def _mm_bias_stats(x, w_t, b):
    n = x.shape[0]
    bn = _row_block(n)
    nb = n // bn

    def body(x_ref, w_ref, b_ref, y_ref, st_ref):
        h = jnp.dot(x_ref[...], w_ref[...],
                    preferred_element_type=jnp.float32) + b_ref[...]
        y_ref[...] = h
        st = jnp.concatenate([jnp.sum(h, axis=0, keepdims=True),
                              jnp.sum(h * h, axis=0, keepdims=True)], axis=0)

        @pl.when(pl.program_id(0) == 0)
        def _():
            st_ref[...] = jnp.zeros_like(st_ref)

        st_ref[...] += st

    return pl.pallas_call(
        body,
        grid=(nb,),
        in_specs=[pl.BlockSpec((bn, D), lambda i: (i, 0)),
                  pl.BlockSpec((D, D), lambda i: (0, 0)),
                  pl.BlockSpec((1, D), lambda i: (0, 0))],
        out_specs=[pl.BlockSpec((bn, D), lambda i: (i, 0)),
                   pl.BlockSpec((2, D), lambda i: (0, 0))],
        out_shape=[jax.ShapeDtypeStruct((n, D), jnp.float32),
                   jax.ShapeDtypeStruct((2, D), jnp.float32)],
    )(x, w_t, b.reshape(1, D))


def _bn_coeffs(st, n, g, be):
    mu = st[0:1] / n
    var = st[1:2] / n - mu * mu
    inv = g / jnp.sqrt(var + BN_EPS)
    return inv, be - mu * inv


def _bnrelu_mm_stats(h, st, g, be, w_t, b):
    n = h.shape[0]
    bn = _row_block(n)
    nb = n // bn

    def body(h_ref, st_in, g_ref, be_ref, w_ref, b_ref, y_ref, st_ref):
        inv, sh = _bn_coeffs(st_in[...], float(n), g_ref[...], be_ref[...])
        z = jnp.maximum(h_ref[...] * inv + sh, 0.0)
        y = jnp.dot(z, w_ref[...], preferred_element_type=jnp.float32) + b_ref[...]
        y_ref[...] = y
        stv = jnp.concatenate([jnp.sum(y, axis=0, keepdims=True),
                               jnp.sum(y * y, axis=0, keepdims=True)], axis=0)

        @pl.when(pl.program_id(0) == 0)
        def _():
            st_ref[...] = jnp.zeros_like(st_ref)

        st_ref[...] += stv

    return pl.pallas_call(
        body,
        grid=(nb,),
        in_specs=[pl.BlockSpec((bn, D), lambda i: (i, 0)),
                  pl.BlockSpec((2, D), lambda i: (0, 0)),
                  pl.BlockSpec((1, D), lambda i: (0, 0)),
                  pl.BlockSpec((1, D), lambda i: (0, 0)),
                  pl.BlockSpec((D, D), lambda i: (0, 0)),
                  pl.BlockSpec((1, D), lambda i: (0, 0))],
        out_specs=[pl.BlockSpec((bn, D), lambda i: (i, 0)),
                   pl.BlockSpec((2, D), lambda i: (0, 0))],
        out_shape=[jax.ShapeDtypeStruct((n, D), jnp.float32),
                   jax.ShapeDtypeStruct((2, D), jnp.float32)],
    )(h, st, g.reshape(1, D), be.reshape(1, D), w_t, b.reshape(1, D))


def _bnrelu_mm_rownorm(h, st, g, be, w_t, b):
    n = h.shape[0]
    bn = _row_block(n)
    nb = n // bn

    def body(h_ref, st_in, g_ref, be_ref, w_ref, b_ref, y_ref):
        inv, sh = _bn_coeffs(st_in[...], float(n), g_ref[...], be_ref[...])
        z = jnp.maximum(h_ref[...] * inv + sh, 0.0)
        y = jnp.dot(z, w_ref[...], preferred_element_type=jnp.float32) + b_ref[...]
        nrm = jnp.sqrt(jnp.sum(y * y, axis=1, keepdims=True))
        y_ref[...] = y / jnp.maximum(nrm, 1e-12)

    return pl.pallas_call(
        body,
        grid=(nb,),
        in_specs=[pl.BlockSpec((bn, D), lambda i: (i, 0)),
                  pl.BlockSpec((2, D), lambda i: (0, 0)),
                  pl.BlockSpec((1, D), lambda i: (0, 0)),
                  pl.BlockSpec((1, D), lambda i: (0, 0)),
                  pl.BlockSpec((D, D), lambda i: (0, 0)),
                  pl.BlockSpec((1, D), lambda i: (0, 0))],
        out_specs=pl.BlockSpec((bn, D), lambda i: (i, 0)),
        out_shape=jax.ShapeDtypeStruct((n, D), jnp.float32),
    )(h, st, g.reshape(1, D), be.reshape(1, D), w_t, b.reshape(1, D))


def _mm_multi(x, w_t_stack):
    n = x.shape[0]
    k = w_t_stack.shape[0]
    bn = _row_block(n)
    nb = n // bn

    def body(x_ref, w_ref, *y_refs):
        xv = x_ref[...]
        wv = w_ref[...]
        for kk in range(k):
            y_refs[kk][...] = jnp.dot(xv, wv[kk],
                                      preferred_element_type=jnp.float32)

    return pl.pallas_call(
        body,
        grid=(nb,),
        in_specs=[pl.BlockSpec((bn, D), lambda i: (i, 0)),
                  pl.BlockSpec((k, D, D), lambda i: (0, 0, 0))],
        out_specs=[pl.BlockSpec((bn, D), lambda i: (i, 0))] * k,
        out_shape=[jax.ShapeDtypeStruct((n, D), jnp.float32)] * k,
    )(x, w_t_stack)


def _combine_pre(ss, cs, x, wr_t, bsum):
    n = x.shape[0]
    r = len(ss)
    bn = _row_block(n)
    nb = n // bn

    def body(*refs):
        s_refs = refs[0:r]
        c_refs = refs[r:2 * r]
        x_ref, w_ref, b_ref, pre_ref, st_ref = refs[2 * r:]
        acc = jnp.dot(x_ref[...], w_ref[...],
                      preferred_element_type=jnp.float32) + b_ref[...]
        for s_ref, c_ref in zip(s_refs, c_refs):
            sv = s_ref[...]
            cv = c_ref[...]
            ssum = sv[0] + sv[1]
            cnt = cv[0, :, 0:1] + cv[1, :, 0:1]
            acc = acc + ssum / jnp.maximum(cnt, 1.0)
        pre_ref[...] = acc
        stv = jnp.concatenate([jnp.sum(acc, axis=0, keepdims=True),
                               jnp.sum(acc * acc, axis=0, keepdims=True)],
                              axis=0)

        @pl.when(pl.program_id(0) == 0)
        def _():
            st_ref[...] = jnp.zeros_like(st_ref)

        st_ref[...] += stv

    in_specs = ([pl.BlockSpec((2, bn, D), lambda i: (0, i, 0))] * r +
                [pl.BlockSpec((2, bn, D), lambda i: (0, i, 0))] * r +
                [pl.BlockSpec((bn, D), lambda i: (i, 0)),
                 pl.BlockSpec((D, D), lambda i: (0, 0)),
                 pl.BlockSpec((1, D), lambda i: (0, 0))])
    return pl.pallas_call(
        body,
        grid=(nb,),
        in_specs=in_specs,
        out_specs=[pl.BlockSpec((bn, D), lambda i: (i, 0)),
                   pl.BlockSpec((2, D), lambda i: (0, 0))],
        out_shape=[jax.ShapeDtypeStruct((n, D), jnp.float32),
                   jax.ShapeDtypeStruct((2, D), jnp.float32)],
    )(*ss, *cs, x, wr_t, bsum.reshape(1, D))


def _bnrelu_emit(h, st, g, be, w_t_stack):
    n = h.shape[0]
    k = w_t_stack.shape[0]
    bn = _row_block(n)
    nb = n // bn

    def body(h_ref, st_in, g_ref, be_ref, w_ref, z_ref, *y_refs):
        inv, sh = _bn_coeffs(st_in[...], float(n), g_ref[...], be_ref[...])
        z = jnp.maximum(h_ref[...] * inv + sh, 0.0)
        z_ref[...] = z
        wv = w_ref[...]
        for kk in range(k):
            y_refs[kk][...] = jnp.dot(z, wv[kk],
                                      preferred_element_type=jnp.float32)

    return pl.pallas_call(
        body,
        grid=(nb,),
        in_specs=[pl.BlockSpec((bn, D), lambda i: (i, 0)),
                  pl.BlockSpec((2, D), lambda i: (0, 0)),
                  pl.BlockSpec((1, D), lambda i: (0, 0)),
                  pl.BlockSpec((1, D), lambda i: (0, 0)),
                  pl.BlockSpec((k, D, D), lambda i: (0, 0, 0))],
        out_specs=[pl.BlockSpec((bn, D), lambda i: (i, 0))] * (1 + k),
        out_shape=[jax.ShapeDtypeStruct((n, D), jnp.float32)] * (1 + k),
    )(h, st, g.reshape(1, D), be.reshape(1, D), w_t_stack)


def _bnrelu_final(h, st, g, be):
    n = h.shape[0]
    bn = _row_block(n)
    nb = n // bn

    def body(h_ref, st_in, g_ref, be_ref, z_ref):
        inv, sh = _bn_coeffs(st_in[...], float(n), g_ref[...], be_ref[...])
        z_ref[...] = jnp.maximum(h_ref[...] * inv + sh, 0.0)

    return pl.pallas_call(
        body,
        grid=(nb,),
        in_specs=[pl.BlockSpec((bn, D), lambda i: (i, 0)),
                  pl.BlockSpec((2, D), lambda i: (0, 0)),
                  pl.BlockSpec((1, D), lambda i: (0, 0)),
                  pl.BlockSpec((1, D), lambda i: (0, 0))],
        out_specs=pl.BlockSpec((bn, D), lambda i: (i, 0)),
        out_shape=jax.ShapeDtypeStruct((n, D), jnp.float32),
    )(h, st, g.reshape(1, D), be.reshape(1, D))


def _seg_geometry(n_dst):
    if n_dst == 50000:
        acc, ch, nchunk = 12800, 12784, 4
    else:
        acc, ch, nchunk = 5120, 5120, 1
    nout = ch * (nchunk - 1) + acc
    return acc, ch, nchunk, nout


def _sc_segsum(y, src, dst, n_dst):
    acc_rows, ch, nchunk, nout = _seg_geometry(n_dst)
    mesh = plsc.VectorSubcoreMesh(core_axis_name="c", subcore_axis_name="s")

    @functools.partial(
        pl.kernel,
        mesh=mesh,
        out_type=jax.ShapeDtypeStruct((NC, nout, D), jnp.float32),
        scratch_types=[
            pltpu.VMEM((EB,), jnp.int32),
            pltpu.VMEM((EB,), jnp.int32),
            pltpu.VMEM((EB, D), jnp.float32),
            pltpu.VMEM_SHARED((acc_rows, D), jnp.float32),
            pltpu.SemaphoreType.DMA,
        ],
    )
    def k(y_hbm, src_hbm, dst_hbm, zeros_hbm, out_hbm, sidx_v, didx_v,
          rows_v, acc_sh, sem):
        cid = lax.axis_index("c")
        sid = lax.axis_index("s")
        wid = sid * NC + cid
        nb_w = (NBATCH - wid + NW - 1) // NW

        for chunk in range(nchunk):
            lo = chunk * ch

            @pl.when(sid == 0)
            def _():
                pltpu.sync_copy(zeros_hbm, acc_sh)

            plsc.subcore_barrier()

            def ebatch(j, c):
                off = (wid + j * NW) * EB
                pltpu.sync_copy(src_hbm.at[pl.ds(off, EB)], sidx_v)
                pltpu.sync_copy(dst_hbm.at[pl.ds(off, EB)], didx_v)
                if nchunk > 1:
                    def fix(i, c2):
                        v = didx_v[pl.ds(i * L, L)]
                        rel = v - lo
                        ok = (rel >= 0) & (rel < ch)
                        didx_v[pl.ds(i * L, L)] = jnp.where(ok, rel, ch)
                        return c2

                    lax.fori_loop(0, EB // L, fix, 0)
                pltpu.async_copy(y_hbm.at[sidx_v], rows_v, sem).wait()
                pltpu.sync_copy(rows_v, acc_sh.at[didx_v], add=True)
                return c

            lax.fori_loop(0, nb_w, ebatch, 0)
            plsc.subcore_barrier()

            @pl.when(sid == 0)
            def _():
                pltpu.sync_copy(acc_sh,
                                out_hbm.at[cid, pl.ds(lo, acc_rows), :])

            plsc.subcore_barrier()

    return k(y, src, dst, jnp.zeros((acc_rows, D), jnp.float32))


def _sc_counts(dst, n_dst):
    acc_rows, ch, nchunk, nout = _seg_geometry(n_dst)
    mesh = plsc.VectorSubcoreMesh(core_axis_name="c", subcore_axis_name="s")

    @functools.partial(
        pl.kernel,
        mesh=mesh,
        out_type=jax.ShapeDtypeStruct((NC, nout, D), jnp.float32),
        scratch_types=[
            pltpu.VMEM((EB,), jnp.int32),
            pltpu.VMEM((EB, D), jnp.float32),
            pltpu.VMEM_SHARED((acc_rows, D), jnp.float32),
        ],
    )
    def k(dst_hbm, ones_hbm, zeros_hbm, out_hbm, didx_v, ones_v, acc_sh):
        cid = lax.axis_index("c")
        sid = lax.axis_index("s")
        wid = sid * NC + cid
        nb_w = (NBATCH - wid + NW - 1) // NW
        pltpu.sync_copy(ones_hbm, ones_v)

        for chunk in range(nchunk):
            lo = chunk * ch

            @pl.when(sid == 0)
            def _():
                pltpu.sync_copy(zeros_hbm, acc_sh)

            plsc.subcore_barrier()

            def ebatch(j, c):
                off = (wid + j * NW) * EB
                pltpu.sync_copy(dst_hbm.at[pl.ds(off, EB)], didx_v)
                if nchunk > 1:
                    def fix(i, c2):
                        v = didx_v[pl.ds(i * L, L)]
                        rel = v - lo
                        ok = (rel >= 0) & (rel < ch)
                        didx_v[pl.ds(i * L, L)] = jnp.where(ok, rel, ch)
                        return c2

                    lax.fori_loop(0, EB // L, fix, 0)
                pltpu.sync_copy(ones_v, acc_sh.at[didx_v], add=True)
                return c

            lax.fori_loop(0, nb_w, ebatch, 0)
            plsc.subcore_barrier()

            @pl.when(sid == 0)
            def _():
                pltpu.sync_copy(acc_sh,
                                out_hbm.at[cid, pl.ds(lo, acc_rows), :])

            plsc.subcore_barrier()

    return k(dst, jnp.ones((EB, D), jnp.float32),
             jnp.zeros((acc_rows, D), jnp.float32))


def kernel(params, src_has_lab, dst_has_lab, src_rev_has_lab, dst_rev_has_lab,
           src_has_diag, dst_has_diag, src_rev_has_diag, dst_rev_has_diag,
           src_has_med, dst_has_med, src_rev_has_med, dst_rev_has_med):
    edges = {
        "has_lab": (src_has_lab, dst_has_lab),
        "rev_has_lab": (src_rev_has_lab, dst_rev_has_lab),
        "has_diag": (src_has_diag, dst_has_diag),
        "rev_has_diag": (src_rev_has_diag, dst_rev_has_diag),
        "has_med": (src_has_med, dst_has_med),
        "rev_has_med": (src_rev_has_med, dst_rev_has_med),
    }
    emb = params["emb"]
    pt = params["pt"]
    convs = params["convs"]
    bns = params["bns"]
    n_of = {nt: emb[nt].shape[0] for nt in NT_ORDER}
    rels_by_src = {"patient": ["has_lab", "has_diag", "has_med"],
                   "lab": ["rev_has_lab"], "diagnosis": ["rev_has_diag"],
                   "medication": ["rev_has_med"]}
    rels_by_dst = {"patient": ["rev_has_lab", "rev_has_diag", "rev_has_med"],
                   "lab": ["has_lab"], "diagnosis": ["has_diag"],
                   "medication": ["has_med"]}

    h1, st1 = _mm_bias_stats(emb["patient"], pt["W1"].T, pt["b1"])
    h2, st2 = _bnrelu_mm_stats(h1, st1, pt["g1"], pt["be1"], pt["W2"].T,
                               pt["b2"])
    hp = _bnrelu_mm_rownorm(h2, st2, pt["g2"], pt["be2"], pt["W3"].T,
                            pt["b3"])

    cnts = {r: _sc_counts(edges[r][1], n_of[REL_DST[r]]) for r in REL_NAMES}

    x = {"patient": hp, "lab": emb["lab"], "diagnosis": emb["diagnosis"],
         "medication": emb["medication"]}

    ys = None
    for layer in range(2):
        cw = convs[layer]
        if ys is None:
            ys = {}
            for st_nt, rl in rels_by_src.items():
                wl = jnp.stack([cw[r]["Wl"].T for r in rl])
                outs = _mm_multi(x[st_nt], wl)
                for r, yv in zip(rl, outs):
                    ys[r] = yv

        segs = {r: _sc_segsum(ys[r], edges[r][0], edges[r][1],
                              n_of[REL_DST[r]]) for r in REL_NAMES}

        pre = {}
        stx = {}
        for nt in NT_ORDER:
            rl = rels_by_dst[nt]
            wr_sum = sum(cw[r]["Wr"] for r in rl)
            bl_sum = sum(cw[r]["bl"] for r in rl)
            pre[nt], stx[nt] = _combine_pre(
                [segs[r] for r in rl], [cnts[r] for r in rl], x[nt],
                wr_sum.T, bl_sum)

        new_x = {}
        if layer == 0:
            nxt = convs[1]
            new_ys = {}
            for nt in NT_ORDER:
                rl = rels_by_src[nt]
                wl = jnp.stack([nxt[r]["Wl"].T for r in rl])
                outs = _bnrelu_emit(pre[nt], stx[nt], bns[0][nt]["g"],
                                    bns[0][nt]["b"], wl)
                new_x[nt] = outs[0]
                for r, yv in zip(rl, outs[1:]):
                    new_ys[r] = yv
            ys = new_ys
        else:
            for nt in NT_ORDER:
                new_x[nt] = _bnrelu_final(pre[nt], stx[nt], bns[1][nt]["g"],
                                          bns[1][nt]["b"])
        x = new_x

    return (x["patient"], x["lab"], x["diagnosis"], x["medication"])
